# baseline (speedup 1.0000x reference)
import functools

import jax
import jax.numpy as jnp
from jax import lax
from jax.experimental import pallas as pl
from jax.experimental.pallas import tpu as pltpu
from jax.experimental.pallas import tpu_sc as plsc

NUM_EMBEDDINGS = 100000
D = 768
CONTEXT_LENGTH = 2048
BATCH = 4
B_TOTAL = BATCH * CONTEXT_LENGTH

NC, NS = 2, 16
NW = NC * NS
POS_PER_W = CONTEXT_LENGTH // NW
CHUNK = 32
HALVES = POS_PER_W // CHUNK
N_CHUNKS = BATCH * HALVES
LANES = 16
VECS_PER_ROW = D // LANES


NBUF = 3


def _embed_body(x_hbm, tok_hbm, pos_hbm, out_hbm, idx_v, rows_v, pos_v,
                sem_g0, sem_g1, sem_g2, sem_o0, sem_o1, sem_o2):
    wid = lax.axis_index("s") * NC + lax.axis_index("c")
    p0 = wid * POS_PER_W

    sem_g = (sem_g0, sem_g1, sem_g2)
    sem_o = (sem_o0, sem_o1, sem_o2)

    for b in range(BATCH):
        pltpu.sync_copy(x_hbm.at[b, pl.ds(p0, POS_PER_W)], idx_v.at[b])
    pltpu.sync_copy(pos_hbm.at[pl.ds(p0, POS_PER_W)], pos_v)

    def gather(c):
        b, h = divmod(c, HALVES)
        return pltpu.async_copy(
            tok_hbm.at[idx_v.at[b, pl.ds(h * CHUNK, CHUNK)]],
            rows_v.at[c % NBUF], sem_g[c % NBUF])

    copies = {0: gather(0), 1: gather(1)}
    out_copies = {}
    for c in range(N_CHUNKS):
        b, h = divmod(c, HALVES)
        if c + 2 < N_CHUNKS:
            if c - 1 >= 0:
                out_copies[c - 1].wait()
            copies[c + 2] = gather(c + 2)
        copies[c].wait()

        buf = rows_v.at[c % NBUF]
        ph = h * CHUNK

        def row_body(r, carry):
            for v in range(VECS_PER_ROW):
                sl = pl.ds(v * LANES, LANES)
                plsc.addupdate(buf.at[r, sl], pos_v[ph + r, sl])
            return carry

        lax.fori_loop(0, CHUNK, row_body, 0)

        row0 = b * CONTEXT_LENGTH + p0 + ph
        out_copies[c] = pltpu.async_copy(
            buf, out_hbm.at[pl.ds(row0, CHUNK)], sem_o[c % NBUF])
    for c in range(max(0, N_CHUNKS - 3), N_CHUNKS):
        out_copies[c].wait()


@jax.jit
def _embed(x_grouped, tok_emb_weight, pos_emb_weight):
    mesh = plsc.VectorSubcoreMesh(
        core_axis_name="c", subcore_axis_name="s", num_cores=NC,
        num_subcores=NS)
    return pl.kernel(
        _embed_body,
        out_type=jax.ShapeDtypeStruct((B_TOTAL, D), jnp.float32),
        mesh=mesh,
        scratch_types=[
            pltpu.VMEM((BATCH, POS_PER_W), jnp.int32),
            pltpu.VMEM((NBUF, CHUNK, D), jnp.float32),
            pltpu.VMEM((POS_PER_W, D), jnp.float32),
            pltpu.SemaphoreType.DMA,
            pltpu.SemaphoreType.DMA,
            pltpu.SemaphoreType.DMA,
            pltpu.SemaphoreType.DMA,
            pltpu.SemaphoreType.DMA,
            pltpu.SemaphoreType.DMA,
        ],
    )(x_grouped, tok_emb_weight, pos_emb_weight)


def kernel(x, tok_emb_weight, pos_emb_weight):
    batch, cxt = x.shape
    out = _embed(x.astype(jnp.int32), tok_emb_weight, pos_emb_weight)
    return out.reshape(batch, cxt, D)


# speedup vs baseline: 1.3282x; 1.3282x over previous
import functools

import jax
import jax.numpy as jnp
from jax import lax
from jax.experimental import pallas as pl
from jax.experimental.pallas import tpu as pltpu
from jax.experimental.pallas import tpu_sc as plsc

NUM_EMBEDDINGS = 100000
D = 768
CONTEXT_LENGTH = 2048
BATCH = 4
B_TOTAL = BATCH * CONTEXT_LENGTH

NC, NS = 2, 16
NW = NC * NS
POS_PER_W = CONTEXT_LENGTH // NW
CHUNK = 32
HALVES = POS_PER_W // CHUNK
N_CHUNKS = BATCH * HALVES
LANES = 16
VECS_PER_ROW = D // LANES


NBUF = 3


def _embed_body(x_hbm, tok_hbm, pos_hbm, out_hbm, idx_v, rows_v,
                sem_g0, sem_g1, sem_g2, sem_o0, sem_o1, sem_o2,
                sem_p0, sem_p1, sem_p2):
    wid = lax.axis_index("s") * NC + lax.axis_index("c")
    p0 = wid * POS_PER_W

    sem_g = (sem_g0, sem_g1, sem_g2)
    sem_o = (sem_o0, sem_o1, sem_o2)
    sem_p = (sem_p0, sem_p1, sem_p2)

    for b in range(BATCH):
        pltpu.sync_copy(x_hbm.at[b, pl.ds(p0, POS_PER_W)], idx_v.at[b])

    def prefill(c):
        h = c % HALVES
        return pltpu.async_copy(
            pos_hbm.at[pl.ds(p0 + h * CHUNK, CHUNK)], rows_v.at[c % NBUF],
            sem_p[c % NBUF])

    def gather(c):
        b, h = divmod(c, HALVES)
        return pltpu.async_copy(
            tok_hbm.at[idx_v.at[b, pl.ds(h * CHUNK, CHUNK)]],
            rows_v.at[c % NBUF], sem_g[c % NBUF], add=True)

    prefills = {0: prefill(0), 1: prefill(1)}
    prefills[0].wait()
    copies = {0: gather(0)}
    out_copies = {}
    for c in range(N_CHUNKS):
        b, h = divmod(c, HALVES)
        if c + 2 < N_CHUNKS:
            if c - 1 >= 0:
                out_copies[c - 1].wait()
            prefills[c + 2] = prefill(c + 2)
        if c + 1 < N_CHUNKS:
            prefills[c + 1].wait()
            copies[c + 1] = gather(c + 1)
        copies[c].wait()

        row0 = b * CONTEXT_LENGTH + p0 + h * CHUNK
        out_copies[c] = pltpu.async_copy(
            rows_v.at[c % NBUF], out_hbm.at[pl.ds(row0, CHUNK)],
            sem_o[c % NBUF])
    for c in range(max(0, N_CHUNKS - 3), N_CHUNKS):
        out_copies[c].wait()


@jax.jit
def _embed(x_grouped, tok_emb_weight, pos_emb_weight):
    mesh = plsc.VectorSubcoreMesh(
        core_axis_name="c", subcore_axis_name="s", num_cores=NC,
        num_subcores=NS)
    return pl.kernel(
        _embed_body,
        out_type=jax.ShapeDtypeStruct((B_TOTAL, D), jnp.float32),
        mesh=mesh,
        scratch_types=[
            pltpu.VMEM((BATCH, POS_PER_W), jnp.int32),
            pltpu.VMEM((NBUF, CHUNK, D), jnp.float32),
            pltpu.SemaphoreType.DMA,
            pltpu.SemaphoreType.DMA,
            pltpu.SemaphoreType.DMA,
            pltpu.SemaphoreType.DMA,
            pltpu.SemaphoreType.DMA,
            pltpu.SemaphoreType.DMA,
            pltpu.SemaphoreType.DMA,
            pltpu.SemaphoreType.DMA,
            pltpu.SemaphoreType.DMA,
        ],
    )(x_grouped, tok_emb_weight, pos_emb_weight)


def kernel(x, tok_emb_weight, pos_emb_weight):
    batch, cxt = x.shape
    out = _embed(x.astype(jnp.int32), tok_emb_weight, pos_emb_weight)
    return out.reshape(batch, cxt, D)
